# baseline (speedup 1.0000x reference)
import functools

import jax
import jax.numpy as jnp
from jax import lax
from jax.experimental import pallas as pl
from jax.experimental.pallas import tpu as pltpu
from jax.experimental.pallas import tpu_sc as plsc

VOCAB = 100000
EMB = 256
TOTAL = 16384

_NC = 2
_NS = 16
_NW = _NC * _NS
_B_PER_W = TOTAL // _NW
_CHUNKS = (32, 64, 128, 128, 128, 32)
_OFFS = tuple(sum(_CHUNKS[:j]) for j in range(len(_CHUNKS)))
_NBUF = 3
_MAXC = max(_CHUNKS)

_mesh = plsc.VectorSubcoreMesh(core_axis_name="c", subcore_axis_name="s")


@functools.partial(
    pl.kernel,
    mesh=_mesh,
    out_type=jax.ShapeDtypeStruct((TOTAL, EMB), jnp.float32),
    scratch_types=[
        pltpu.VMEM((_B_PER_W,), jnp.int32),
    ]
    + [pltpu.VMEM((_MAXC, EMB), jnp.float32) for _ in range(_NBUF)]
    + [pltpu.SemaphoreType.DMA for _ in range(2 * _NBUF + 1)],
)
def _gather_kernel(tokens_hbm, table_hbm, out_hbm, idx_v, *bufs_sems):
    bufs = bufs_sems[:_NBUF]
    gsems = bufs_sems[_NBUF : 2 * _NBUF]
    wsems = bufs_sems[2 * _NBUF : 3 * _NBUF]
    isem = bufs_sems[3 * _NBUF]
    wid = lax.axis_index("s") * _NC + lax.axis_index("c")
    base = wid * _B_PER_W

    c0 = _CHUNKS[0]
    pltpu.sync_copy(tokens_hbm.at[pl.ds(base, c0)], idx_v.at[pl.ds(0, c0)])
    n = len(_CHUNKS)
    gcp = [None] * _NBUF
    wcp = [None] * _NBUF
    icp = None
    for j in range(n):
        b = j % _NBUF
        if j >= _NBUF:
            wcp[b].wait()
        if j == 1:
            icp.wait()
        gcp[b] = pltpu.async_copy(
            table_hbm.at[idx_v.at[pl.ds(_OFFS[j], _CHUNKS[j])]],
            bufs[b].at[pl.ds(0, _CHUNKS[j])],
            gsems[b],
        )
        if j == 0:
            icp = pltpu.async_copy(
                tokens_hbm.at[pl.ds(base + c0, _B_PER_W - c0)],
                idx_v.at[pl.ds(c0, _B_PER_W - c0)],
                isem,
            )
        d = j - (_NBUF - 1)
        if d >= 0:
            db = d % _NBUF
            gcp[db].wait()
            wcp[db] = pltpu.async_copy(
                bufs[db].at[pl.ds(0, _CHUNKS[d])],
                out_hbm.at[pl.ds(base + _OFFS[d], _CHUNKS[d])],
                wsems[db],
            )
    for d in range(max(0, n - (_NBUF - 1)), n):
        db = d % _NBUF
        gcp[db].wait()
        wcp[db] = pltpu.async_copy(
            bufs[db].at[pl.ds(0, _CHUNKS[d])],
            out_hbm.at[pl.ds(base + _OFFS[d], _CHUNKS[d])],
            wsems[db],
        )
    for d in range(max(0, n - _NBUF), n):
        wcp[d % _NBUF].wait()


def kernel(flat_tokens, cu_seqlens, table):
    del cu_seqlens
    return _gather_kernel(flat_tokens, table)


# speedup vs baseline: 1.0057x; 1.0057x over previous
import functools

import jax
import jax.numpy as jnp
from jax import lax
from jax.experimental import pallas as pl
from jax.experimental.pallas import tpu as pltpu
from jax.experimental.pallas import tpu_sc as plsc

VOCAB = 100000
EMB = 256
TOTAL = 16384

_NC = 2
_NS = 16
_NW = _NC * _NS
_B_PER_W = TOTAL // _NW
_CHUNKS = (128, 128, 128, 64, 32, 32)
_OFFS = tuple(sum(_CHUNKS[:j]) for j in range(len(_CHUNKS)))
_NBUF = 3
_MAXC = max(_CHUNKS)

_mesh = plsc.VectorSubcoreMesh(core_axis_name="c", subcore_axis_name="s")


@functools.partial(
    pl.kernel,
    mesh=_mesh,
    out_type=jax.ShapeDtypeStruct((TOTAL, EMB), jnp.float32),
    scratch_types=[
        pltpu.VMEM((_B_PER_W,), jnp.int32),
    ]
    + [pltpu.VMEM((_MAXC, EMB), jnp.float32) for _ in range(_NBUF)]
    + [pltpu.SemaphoreType.DMA for _ in range(2 * _NBUF + 1)],
)
def _gather_kernel(tokens_hbm, table_hbm, out_hbm, idx_v, *bufs_sems):
    bufs = bufs_sems[:_NBUF]
    gsems = bufs_sems[_NBUF : 2 * _NBUF]
    wsems = bufs_sems[2 * _NBUF : 3 * _NBUF]
    isem = bufs_sems[3 * _NBUF]
    del isem
    wid = lax.axis_index("s") * _NC + lax.axis_index("c")
    base = wid * _B_PER_W
    pltpu.sync_copy(tokens_hbm.at[pl.ds(base, _B_PER_W)], idx_v)

    n = len(_CHUNKS)
    gcp = [None] * _NBUF
    wcp = [None] * _NBUF
    for j in range(n):
        b = j % _NBUF
        if j >= _NBUF:
            wcp[b].wait()
        gcp[b] = pltpu.async_copy(
            table_hbm.at[idx_v.at[pl.ds(_OFFS[j], _CHUNKS[j])]],
            bufs[b].at[pl.ds(0, _CHUNKS[j])],
            gsems[b],
        )
        d = j - 1
        if d >= 0:
            db = d % _NBUF
            gcp[db].wait()
            wcp[db] = pltpu.async_copy(
                bufs[db].at[pl.ds(0, _CHUNKS[d])],
                out_hbm.at[pl.ds(base + _OFFS[d], _CHUNKS[d])],
                wsems[db],
            )
    d = n - 1
    db = d % _NBUF
    gcp[db].wait()
    wcp[db] = pltpu.async_copy(
        bufs[db].at[pl.ds(0, _CHUNKS[d])],
        out_hbm.at[pl.ds(base + _OFFS[d], _CHUNKS[d])],
        wsems[db],
    )
    for d in range(max(0, n - _NBUF), n):
        wcp[d % _NBUF].wait()


def kernel(flat_tokens, cu_seqlens, table):
    del cu_seqlens
    return _gather_kernel(flat_tokens, table)


# speedup vs baseline: 1.0214x; 1.0155x over previous
import functools

import jax
import jax.numpy as jnp
from jax import lax
from jax.experimental import pallas as pl
from jax.experimental.pallas import tpu as pltpu
from jax.experimental.pallas import tpu_sc as plsc

VOCAB = 100000
EMB = 256
TOTAL = 16384

_NC = 2
_NS = 16
_NW = _NC * _NS
_B_PER_W = TOTAL // _NW
_CHUNKS = (128, 128, 64, 64, 64, 32, 16, 16)
_OFFS = tuple(sum(_CHUNKS[:j]) for j in range(len(_CHUNKS)))
_N = len(_CHUNKS)
_BIDX = (0, 1, 2, 3, 4, 5, 6, 0)
_NBUF = 7
_BUFSZ = _CHUNKS[:_NBUF]
_DIST = 3

_mesh = plsc.VectorSubcoreMesh(core_axis_name="c", subcore_axis_name="s")


@functools.partial(
    pl.kernel,
    mesh=_mesh,
    out_type=jax.ShapeDtypeStruct((TOTAL, EMB), jnp.float32),
    scratch_types=[
        pltpu.VMEM((_B_PER_W,), jnp.int32),
    ]
    + [pltpu.VMEM((_BUFSZ[b], EMB), jnp.float32) for b in range(_NBUF)]
    + [pltpu.SemaphoreType.DMA for _ in range(2 * _N)],
)
def _gather_kernel(tokens_hbm, table_hbm, out_hbm, idx_v, *bufs_sems):
    bufs = bufs_sems[:_NBUF]
    gsems = bufs_sems[_NBUF : _NBUF + _N]
    wsems = bufs_sems[_NBUF + _N :]
    wid = lax.axis_index("s") * _NC + lax.axis_index("c")
    base = wid * _B_PER_W
    pltpu.sync_copy(tokens_hbm.at[pl.ds(base, _B_PER_W)], idx_v)

    gcp = [None] * _N
    wcp = [None] * _N

    def issue_write(d):
        wcp[d] = pltpu.async_copy(
            bufs[_BIDX[d]].at[pl.ds(0, _CHUNKS[d])],
            out_hbm.at[pl.ds(base + _OFFS[d], _CHUNKS[d])],
            wsems[d],
        )

    for j in range(_N):
        if j == _N - 1:
            wcp[0].wait()
        gcp[j] = pltpu.async_copy(
            table_hbm.at[idx_v.at[pl.ds(_OFFS[j], _CHUNKS[j])]],
            bufs[_BIDX[j]].at[pl.ds(0, _CHUNKS[j])],
            gsems[j],
        )
        d = j - _DIST
        if d >= 0:
            gcp[d].wait()
            issue_write(d)
    for d in range(_N - _DIST, _N):
        gcp[d].wait()
        issue_write(d)
    for d in range(1, _N):
        wcp[d].wait()


def kernel(flat_tokens, cu_seqlens, table):
    del cu_seqlens
    return _gather_kernel(flat_tokens, table)
